# baseline (speedup 1.0000x reference)
import functools
import jax
import jax.numpy as jnp
from jax import lax
from jax.experimental import pallas as pl
from jax.experimental.pallas import tpu as pltpu
from jax.experimental.pallas import tpu_sc as plsc

N = 10000
E = 320000
D_IN = 128
D_HID = 128
N_CLS = 40
D2P = 48

NC = 2
NS = 16
NW = NC * NS
E_PER_W = E // NW
E_PER_W_P = 10240
CH = 128
NCHUNK = E_PER_W_P // CH
KD = 5
GSZD = KD * CH
NGD = E_PER_W_P // GSZD
N_PAD = 10240
ROWS_PER_TILE = N_PAD // NS


def _sc_mesh():
    return plsc.VectorSubcoreMesh(core_axis_name="c", subcore_axis_name="s")


_SC_PARAMS = pltpu.CompilerParams(use_tc_tiling_on_sc=False)


def _make_agg(D):
    BL = 20

    def body_fn(y_hbm, src_hbm, dst_hbm, zeros_hbm, out_hbm,
                srcb0, srcb1, dstb0, dstb1, rows0, rows1,
                acc_sh, gsem0, gsem1, ssem0, ssem1):
        c = lax.axis_index("c")
        s = lax.axis_index("s")
        wid = c * NS + s
        base = wid * E_PER_W_P
        rbase = s * ROWS_PER_TILE

        srcb = (srcb0, srcb1)
        dstb = (dstb0, dstb1)
        rows = (rows0, rows1)
        gsem = (gsem0, gsem1)
        ssem = (ssem0, ssem1)

        NBLK = E_PER_W_P // (BL * CH)

        def idx_load(blk, B):
            off = base + blk * BL * CH
            pltpu.sync_copy(src_hbm.at[pl.ds(off, BL * CH)], srcb[B])
            pltpu.sync_copy(dst_hbm.at[wid * NBLK + blk], dstb[B])

        def fire_g(t, B, b):
            pltpu.async_copy(y_hbm.at[srcb[B].at[pl.ds(t * CH, CH)]],
                             rows[b], gsem[b])

        def wait_g(b):
            pltpu.make_async_copy(y_hbm.at[srcb[0].at[pl.ds(0, CH)]],
                                  rows[b], gsem[b]).wait()

        def wait_s(b):
            pltpu.make_async_copy(rows[b], acc_sh.at[dstb[0].at[0]],
                                  ssem[b]).wait()

        def chunk(t, B, b, drain=True, pre=True):
            wait_g(b)
            pltpu.async_copy(rows[b], acc_sh.at[dstb[B].at[t]], ssem[b],
                             add=True)
            if drain:
                wait_s(1 - b)
            if pre:
                fire_g(t + 1, B, 1 - b)

        idx_load(0, 0)
        fire_g(0, 0, 0)
        pltpu.sync_copy(zeros_hbm.at[pl.ds(rbase, ROWS_PER_TILE)],
                        acc_sh.at[pl.ds(rbase, ROWS_PER_TILE)])
        plsc.subcore_barrier()

        for blk in range(NBLK):
            B = blk % 2
            p = 0
            if blk == 0:
                chunk(0, B, 0, drain=False)
                chunk(1, B, 1)
                o = 2
            else:
                idx_load(blk, B)
                fire_g(0, B, p)
                chunk(0, B, p)
                o = 1

            def pair(i, carry):
                chunk(o + 2 * i, B, (p + o) % 2)
                chunk(o + 2 * i + 1, B, (p + o + 1) % 2)
                return carry

            npair = (BL - o - 1) // 2
            lax.fori_loop(0, npair, pair, 0)
            for t in range(o + 2 * npair, BL - 1):
                chunk(t, B, (p + t) % 2)
            chunk(BL - 1, B, (p + BL - 1) % 2, pre=False)

        wait_s((BL - 1) % 2)
        plsc.subcore_barrier()
        pltpu.sync_copy(acc_sh.at[pl.ds(rbase, ROWS_PER_TILE)],
                        out_hbm.at[c, pl.ds(rbase, ROWS_PER_TILE)])

    return pl.kernel(
        body_fn,
        mesh=_sc_mesh(),
        compiler_params=_SC_PARAMS,
        out_type=jax.ShapeDtypeStruct((NC, N_PAD, D), jnp.float32),
        scratch_types=(
            [pltpu.VMEM((BL * CH,), jnp.int32)] * 2
            + [pltpu.VMEM((BL, CH), jnp.int32)] * 2
            + [pltpu.VMEM((CH, D), jnp.float32)] * 2
            + [pltpu.VMEM_SHARED((N_PAD, D), jnp.float32)]
            + [pltpu.SemaphoreType.DMA] * 4
        ),
    )


_agg128 = _make_agg(D_HID)
_agg48 = _make_agg(D2P)


@functools.partial(
    pl.kernel,
    mesh=_sc_mesh(),
    compiler_params=_SC_PARAMS,
    out_type=jax.ShapeDtypeStruct((NC, N_PAD, 16), jnp.float32),
    scratch_types=(
        [pltpu.VMEM((CH, 16), jnp.float32)]
        + [pltpu.VMEM((CH,), jnp.int32)] * (2 * KD)
        + [pltpu.VMEM_SHARED((N_PAD, 16), jnp.float32)]
        + [pltpu.SemaphoreType.DMA] * 2
    ),
)
def _deg_pass(dst_hbm, ones_hbm, zeros_hbm, out_hbm,
              ones_v, d00, d01, d02, d03, d04, d10, d11, d12, d13, d14,
              acc_sh, ssem0, ssem1):
    c = lax.axis_index("c")
    s = lax.axis_index("s")
    wid = c * NS + s
    base = wid * E_PER_W_P
    rbase = s * ROWS_PER_TILE

    dstb = ((d00, d01, d02, d03, d04), (d10, d11, d12, d13, d14))
    ssem = (ssem0, ssem1)

    def idx_load(g, b):
        off = base + g * GSZD
        for k in range(KD):
            pltpu.sync_copy(dst_hbm.at[pl.ds(off + k * CH, CH)], dstb[b][k])

    def fire_s(b):
        for k in range(KD):
            pltpu.async_copy(ones_v, acc_sh.at[dstb[b][k]], ssem[b],
                             add=True)

    def wait_s(b):
        for k in range(KD):
            pltpu.make_async_copy(ones_v, acc_sh.at[dstb[b][k]],
                                  ssem[b]).wait()

    pltpu.sync_copy(ones_hbm, ones_v)
    idx_load(0, 0)
    pltpu.sync_copy(zeros_hbm.at[pl.ds(rbase, ROWS_PER_TILE)],
                    acc_sh.at[pl.ds(rbase, ROWS_PER_TILE)])
    plsc.subcore_barrier()

    fire_s(0)
    idx_load(1, 1)

    def group(g, b):
        nb = 1 - b
        fire_s(b)
        wait_s(nb)
        idx_load(g + 1, nb)

    group(1, 1)

    def pair(t, carry):
        group(2 * t + 2, 0)
        group(2 * t + 3, 1)
        return carry

    lax.fori_loop(0, (NGD - 4) // 2, pair, 0)
    group(NGD - 2, 0)
    fire_s(1)
    wait_s(0)
    wait_s(1)
    plsc.subcore_barrier()
    pltpu.sync_copy(acc_sh.at[pl.ds(rbase, ROWS_PER_TILE)],
                    out_hbm.at[c, pl.ds(rbase, ROWS_PER_TILE)])


_BLK = 1000


def _xw_body(x_ref, w1_ref, xw_ref):
    xw_ref[...] = jnp.dot(x_ref[...], w1_ref[...],
                          preferred_element_type=jnp.float32)


def _scale_body(parts_ref, xw_ref, dinv_ref, y1_ref):
    deg = parts_ref[0][:, :1] + parts_ref[1][:, :1] + 1.0
    dinv = lax.rsqrt(jnp.clip(deg, 1.0, None))
    dinv_ref[...] = jnp.broadcast_to(dinv, (_BLK, 16))
    y1_ref[...] = xw_ref[...] * dinv


def _layer2_body(p_ref, y1_ref, dinv_ref, b1_ref, w2_ref, y2_ref):
    dinv = dinv_ref[:, :1]
    z = p_ref[0] + p_ref[1] + y1_ref[...]
    h = jnp.maximum(z * dinv + b1_ref[...], 0.0)
    y2_ref[...] = jnp.dot(h, w2_ref[...],
                          preferred_element_type=jnp.float32) * dinv


def _epilogue_body(p_ref, y2_ref, dinv_ref, b2_ref, o_ref):
    dinv = dinv_ref[:, :1]
    z = p_ref[0] + p_ref[1] + y2_ref[...]
    o_ref[...] = z * dinv + b2_ref[...]


def kernel(x, edge_index, W1, b1, W2, b2):
    f32 = jnp.float32
    pad = E_PER_W_P - E_PER_W
    src = jnp.concatenate(
        [edge_index[0].reshape(NW, E_PER_W),
         jnp.zeros((NW, pad), jnp.int32)], axis=1).reshape(-1)
    dst = jnp.concatenate(
        [edge_index[1].reshape(NW, E_PER_W),
         jnp.full((NW, pad), N_PAD - 1, jnp.int32)], axis=1).reshape(-1)
    ones16 = jnp.ones((CH, 16), f32)
    zeros16 = jnp.zeros((N_PAD, 16), f32)
    zeros48 = jnp.zeros((N_PAD, D2P), f32)
    zeros128 = jnp.zeros((N_PAD, D_HID), f32)
    W2p = jnp.zeros((D_HID, D2P), f32).at[:, :N_CLS].set(W2)
    b1r = b1.reshape(1, D_HID)
    b2p = jnp.zeros((1, D2P), f32).at[0, :N_CLS].set(b2)

    grid = N // _BLK

    deg_parts = _deg_pass(dst, ones16, zeros16)

    xw1 = pl.pallas_call(
        _xw_body,
        grid=(grid,),
        in_specs=[pl.BlockSpec((_BLK, D_IN), lambda i: (i, 0)),
                  pl.BlockSpec((D_IN, D_HID), lambda i: (0, 0))],
        out_specs=pl.BlockSpec((_BLK, D_HID), lambda i: (i, 0)),
        out_shape=jax.ShapeDtypeStruct((N, D_HID), f32),
    )(x, W1)

    dinv16, y1 = pl.pallas_call(
        _scale_body,
        grid=(grid,),
        in_specs=[pl.BlockSpec((NC, _BLK, 16), lambda i: (0, i, 0)),
                  pl.BlockSpec((_BLK, D_HID), lambda i: (i, 0))],
        out_specs=[pl.BlockSpec((_BLK, 16), lambda i: (i, 0)),
                   pl.BlockSpec((_BLK, D_HID), lambda i: (i, 0))],
        out_shape=[jax.ShapeDtypeStruct((N, 16), f32),
                   jax.ShapeDtypeStruct((N, D_HID), f32)],
    )(deg_parts, xw1)

    dst3 = dst.reshape(NW * 4, 20, 128)

    p1 = _agg128(y1, src, dst3, zeros128)

    y2 = pl.pallas_call(
        _layer2_body,
        grid=(grid,),
        in_specs=[pl.BlockSpec((NC, _BLK, D_HID), lambda i: (0, i, 0)),
                  pl.BlockSpec((_BLK, D_HID), lambda i: (i, 0)),
                  pl.BlockSpec((_BLK, 16), lambda i: (i, 0)),
                  pl.BlockSpec((1, D_HID), lambda i: (0, 0)),
                  pl.BlockSpec((D_HID, D2P), lambda i: (0, 0))],
        out_specs=pl.BlockSpec((_BLK, D2P), lambda i: (i, 0)),
        out_shape=jax.ShapeDtypeStruct((N, D2P), f32),
    )(p1, y1, dinv16, b1r, W2p)

    p2 = _agg48(y2, src, dst3, zeros48)

    outp = pl.pallas_call(
        _epilogue_body,
        grid=(grid,),
        in_specs=[pl.BlockSpec((NC, _BLK, D2P), lambda i: (0, i, 0)),
                  pl.BlockSpec((_BLK, D2P), lambda i: (i, 0)),
                  pl.BlockSpec((_BLK, 16), lambda i: (i, 0)),
                  pl.BlockSpec((1, D2P), lambda i: (0, 0))],
        out_specs=pl.BlockSpec((_BLK, D2P), lambda i: (i, 0)),
        out_shape=jax.ShapeDtypeStruct((N, D2P), f32),
    )(p2, y2, dinv16, b2p)

    return outp[:, :N_CLS]


# speedup vs baseline: 1.7455x; 1.7455x over previous
import functools
import jax
import jax.numpy as jnp
from jax import lax
from jax.experimental import pallas as pl
from jax.experimental.pallas import tpu as pltpu
from jax.experimental.pallas import tpu_sc as plsc

N = 10000
E = 320000
D_IN = 128
D_HID = 128
N_CLS = 40
D2P = 48

NC = 2
NS = 16
NW = NC * NS
E_PER_W = E // NW
CH = 80
KD = 5
GSZD = KD * CH
NGD = E_PER_W // GSZD
N_PAD = 10240
ROWS_PER_TILE = N_PAD // NS


def _sc_mesh():
    return plsc.VectorSubcoreMesh(core_axis_name="c", subcore_axis_name="s")


_SC_PARAMS = pltpu.CompilerParams(use_tc_tiling_on_sc=False)


def _make_agg(D):
    BL = 25

    def body_fn(y_hbm, src_hbm, dst_hbm, zeros_hbm, out_hbm,
                srcb0, srcb1, dstb0, dstb1, rows0, rows1,
                acc_sh, gsem0, gsem1, ssem0, ssem1):
        c = lax.axis_index("c")
        s = lax.axis_index("s")
        wid = c * NS + s
        base = wid * E_PER_W
        rbase = s * ROWS_PER_TILE

        srcb = (srcb0, srcb1)
        dstb = (dstb0, dstb1)
        rows = (rows0, rows1)
        gsem = (gsem0, gsem1)
        ssem = (ssem0, ssem1)

        NBLK = E_PER_W // (BL * CH)

        def idx_load(blk, B):
            off = base + blk * BL * CH
            pltpu.sync_copy(src_hbm.at[pl.ds(off, BL * CH)], srcb[B])
            pltpu.sync_copy(dst_hbm.at[wid * NBLK + blk], dstb[B])

        def fire_g(t, B, b):
            pltpu.async_copy(y_hbm.at[srcb[B].at[pl.ds(t * CH, CH)]],
                             rows[b], gsem[b])

        def wait_g(b):
            pltpu.make_async_copy(y_hbm.at[srcb[0].at[pl.ds(0, CH)]],
                                  rows[b], gsem[b]).wait()

        def wait_s(b):
            pltpu.make_async_copy(rows[b], acc_sh.at[dstb[0].at[0]],
                                  ssem[b]).wait()

        def chunk(t, B, b, drain=True, pre=True):
            wait_g(b)
            pltpu.async_copy(rows[b], acc_sh.at[dstb[B].at[t]], ssem[b],
                             add=True)
            if drain:
                wait_s(1 - b)
            if pre:
                fire_g(t + 1, B, 1 - b)

        idx_load(0, 0)
        fire_g(0, 0, 0)
        pltpu.sync_copy(zeros_hbm.at[pl.ds(rbase, ROWS_PER_TILE)],
                        acc_sh.at[pl.ds(rbase, ROWS_PER_TILE)])
        plsc.subcore_barrier()

        for blk in range(NBLK):
            B = p = blk % 2
            if blk == 0:
                chunk(0, B, 0, drain=False)
                chunk(1, B, 1)
                o = 2
            else:
                idx_load(blk, B)
                fire_g(0, B, p)
                chunk(0, B, p)
                o = 1

            def pair(i, carry):
                chunk(o + 2 * i, B, (p + o) % 2)
                chunk(o + 2 * i + 1, B, (p + o + 1) % 2)
                return carry

            npair = (BL - o - 1) // 2
            lax.fori_loop(0, npair, pair, 0)
            for t in range(o + 2 * npair, BL - 1):
                chunk(t, B, (p + t) % 2)
            chunk(BL - 1, B, (p + BL - 1) % 2, pre=False)

        wait_s(0)
        plsc.subcore_barrier()
        pltpu.sync_copy(acc_sh.at[pl.ds(rbase, ROWS_PER_TILE)],
                        out_hbm.at[c, pl.ds(rbase, ROWS_PER_TILE)])

    return pl.kernel(
        body_fn,
        mesh=_sc_mesh(),
        compiler_params=_SC_PARAMS,
        out_type=jax.ShapeDtypeStruct((NC, N_PAD, D), jnp.float32),
        scratch_types=(
            [pltpu.VMEM((BL * CH,), jnp.int32)] * 2
            + [pltpu.VMEM((BL, CH), jnp.int32)] * 2
            + [pltpu.VMEM((CH, D), jnp.float32)] * 2
            + [pltpu.VMEM_SHARED((N_PAD, D), jnp.float32)]
            + [pltpu.SemaphoreType.DMA] * 4
        ),
    )


_agg128 = _make_agg(D_HID)
_agg48 = _make_agg(D2P)


@functools.partial(
    pl.kernel,
    mesh=_sc_mesh(),
    compiler_params=_SC_PARAMS,
    out_type=jax.ShapeDtypeStruct((NC, N_PAD, 16), jnp.float32),
    scratch_types=(
        [pltpu.VMEM((CH, 16), jnp.float32)]
        + [pltpu.VMEM((CH,), jnp.int32)] * (2 * KD)
        + [pltpu.VMEM_SHARED((N_PAD, 16), jnp.float32)]
        + [pltpu.SemaphoreType.DMA] * 2
    ),
)
def _deg_pass(dst_hbm, ones_hbm, zeros_hbm, out_hbm,
              ones_v, d00, d01, d02, d03, d04, d10, d11, d12, d13, d14,
              acc_sh, ssem0, ssem1):
    c = lax.axis_index("c")
    s = lax.axis_index("s")
    wid = c * NS + s
    base = wid * E_PER_W
    rbase = s * ROWS_PER_TILE

    dstb = ((d00, d01, d02, d03, d04), (d10, d11, d12, d13, d14))
    ssem = (ssem0, ssem1)

    def idx_load(g, b):
        off = base + g * GSZD
        for k in range(KD):
            pltpu.sync_copy(dst_hbm.at[pl.ds(off + k * CH, CH)], dstb[b][k])

    def fire_s(b):
        for k in range(KD):
            pltpu.async_copy(ones_v, acc_sh.at[dstb[b][k]], ssem[b],
                             add=True)

    def wait_s(b):
        for k in range(KD):
            pltpu.make_async_copy(ones_v, acc_sh.at[dstb[b][k]],
                                  ssem[b]).wait()

    pltpu.sync_copy(ones_hbm, ones_v)
    idx_load(0, 0)
    pltpu.sync_copy(zeros_hbm.at[pl.ds(rbase, ROWS_PER_TILE)],
                    acc_sh.at[pl.ds(rbase, ROWS_PER_TILE)])
    plsc.subcore_barrier()

    fire_s(0)
    idx_load(1, 1)

    def group(g, b):
        nb = 1 - b
        fire_s(b)
        wait_s(nb)
        idx_load(g + 1, nb)

    group(1, 1)

    def pair(t, carry):
        group(2 * t + 2, 0)
        group(2 * t + 3, 1)
        return carry

    lax.fori_loop(0, (NGD - 3) // 2, pair, 0)

    fire_s(0)
    wait_s(1)
    wait_s(0)
    plsc.subcore_barrier()
    pltpu.sync_copy(acc_sh.at[pl.ds(rbase, ROWS_PER_TILE)],
                    out_hbm.at[c, pl.ds(rbase, ROWS_PER_TILE)])


_BLK = 1000


def _y1_body(parts_ref, x_ref, w1_ref, dinv_ref, y1_ref):
    deg = parts_ref[0][:, :1] + parts_ref[1][:, :1] + 1.0
    dinv = lax.rsqrt(jnp.clip(deg, 1.0, None))
    dinv_ref[...] = jnp.broadcast_to(dinv, (_BLK, 16))
    y1_ref[...] = jnp.dot(x_ref[...], w1_ref[...],
                          preferred_element_type=jnp.float32) * dinv


def _layer2_body(p_ref, y1_ref, dinv_ref, b1_ref, w2_ref, y2_ref):
    dinv = dinv_ref[:, :1]
    z = p_ref[0] + p_ref[1] + y1_ref[...]
    h = jnp.maximum(z * dinv + b1_ref[...], 0.0)
    y2_ref[...] = jnp.dot(h, w2_ref[...],
                          preferred_element_type=jnp.float32) * dinv


def _epilogue_body(p_ref, y2_ref, dinv_ref, b2_ref, o_ref):
    dinv = dinv_ref[:, :1]
    z = p_ref[0] + p_ref[1] + y2_ref[...]
    o_ref[...] = z * dinv + b2_ref[...]


def kernel(x, edge_index, W1, b1, W2, b2):
    f32 = jnp.float32
    src = edge_index[0]
    dst = edge_index[1]
    ones16 = jnp.ones((CH, 16), f32)
    zeros16 = jnp.zeros((N_PAD, 16), f32)
    zeros48 = jnp.zeros((N_PAD, D2P), f32)
    zeros128 = jnp.zeros((N_PAD, D_HID), f32)
    W2p = jnp.zeros((D_HID, D2P), f32).at[:, :N_CLS].set(W2)
    b1r = b1.reshape(1, D_HID)
    b2p = jnp.zeros((1, D2P), f32).at[0, :N_CLS].set(b2)

    grid = N // _BLK

    deg_parts = _deg_pass(dst, ones16, zeros16)

    dinv16, y1 = pl.pallas_call(
        _y1_body,
        grid=(grid,),
        in_specs=[pl.BlockSpec((NC, _BLK, 16), lambda i: (0, i, 0)),
                  pl.BlockSpec((_BLK, D_IN), lambda i: (i, 0)),
                  pl.BlockSpec((D_IN, D_HID), lambda i: (0, 0))],
        out_specs=[pl.BlockSpec((_BLK, 16), lambda i: (i, 0)),
                   pl.BlockSpec((_BLK, D_HID), lambda i: (i, 0))],
        out_shape=[jax.ShapeDtypeStruct((N, 16), f32),
                   jax.ShapeDtypeStruct((N, D_HID), f32)],
    )(deg_parts, x, W1)

    dst3 = dst.reshape(NW * (E_PER_W // 2000), 25, 80)

    p1 = _agg128(y1, src, dst3, zeros128)

    y2 = pl.pallas_call(
        _layer2_body,
        grid=(grid,),
        in_specs=[pl.BlockSpec((NC, _BLK, D_HID), lambda i: (0, i, 0)),
                  pl.BlockSpec((_BLK, D_HID), lambda i: (i, 0)),
                  pl.BlockSpec((_BLK, 16), lambda i: (i, 0)),
                  pl.BlockSpec((1, D_HID), lambda i: (0, 0)),
                  pl.BlockSpec((D_HID, D2P), lambda i: (0, 0))],
        out_specs=pl.BlockSpec((_BLK, D2P), lambda i: (i, 0)),
        out_shape=jax.ShapeDtypeStruct((N, D2P), f32),
    )(p1, y1, dinv16, b1r, W2p)

    p2 = _agg48(y2, src, dst3, zeros48)

    outp = pl.pallas_call(
        _epilogue_body,
        grid=(grid,),
        in_specs=[pl.BlockSpec((NC, _BLK, D2P), lambda i: (0, i, 0)),
                  pl.BlockSpec((_BLK, D2P), lambda i: (i, 0)),
                  pl.BlockSpec((_BLK, 16), lambda i: (i, 0)),
                  pl.BlockSpec((1, D2P), lambda i: (0, 0))],
        out_specs=pl.BlockSpec((_BLK, D2P), lambda i: (i, 0)),
        out_shape=jax.ShapeDtypeStruct((N, D2P), f32),
    )(p2, y2, dinv16, b2p)

    return outp[:, :N_CLS]


# speedup vs baseline: 1.7813x; 1.0205x over previous
import functools
import jax
import jax.numpy as jnp
from jax import lax
from jax.experimental import pallas as pl
from jax.experimental.pallas import tpu as pltpu
from jax.experimental.pallas import tpu_sc as plsc

N = 10000
E = 320000
D_IN = 128
D_HID = 128
N_CLS = 40
D2P = 48

NC = 2
NS = 16
NW = NC * NS
E_PER_W = E // NW
CH = 80
KD = 5
GSZD = KD * CH
NGD = E_PER_W // GSZD
N_PAD = 10240
ROWS_PER_TILE = N_PAD // NS


def _sc_mesh():
    return plsc.VectorSubcoreMesh(core_axis_name="c", subcore_axis_name="s")


_SC_PARAMS = pltpu.CompilerParams(use_tc_tiling_on_sc=False)


def _make_agg(D):
    NCH = E_PER_W // CH

    def body_fn(y_hbm, src_hbm, dst_hbm, zeros_hbm, out_hbm,
                srcb, dstb, rows0, rows1,
                acc_sh, gsem0, gsem1, ssem0, ssem1):
        c = lax.axis_index("c")
        s = lax.axis_index("s")
        wid = c * NS + s
        base = wid * E_PER_W
        rbase = s * ROWS_PER_TILE

        rows = (rows0, rows1)
        gsem = (gsem0, gsem1)
        ssem = (ssem0, ssem1)

        def fire_g(t, b):
            pltpu.async_copy(y_hbm.at[srcb.at[pl.ds(t * CH, CH)]],
                             rows[b], gsem[b])

        def wait_g(b):
            pltpu.make_async_copy(y_hbm.at[srcb.at[pl.ds(0, CH)]],
                                  rows[b], gsem[b]).wait()

        def wait_s(b):
            pltpu.make_async_copy(rows[b], acc_sh.at[dstb.at[0]],
                                  ssem[b]).wait()

        def chunk(t, b, drain=True, pre=True):
            wait_g(b)
            pltpu.async_copy(rows[b], acc_sh.at[dstb.at[t]], ssem[b],
                             add=True)
            if drain:
                wait_s(1 - b)
            if pre:
                fire_g(t + 1, 1 - b)

        pltpu.sync_copy(src_hbm.at[pl.ds(base, E_PER_W)], srcb)
        pltpu.sync_copy(dst_hbm.at[wid], dstb)
        fire_g(0, 0)
        pltpu.sync_copy(zeros_hbm.at[pl.ds(rbase, ROWS_PER_TILE)],
                        acc_sh.at[pl.ds(rbase, ROWS_PER_TILE)])
        plsc.subcore_barrier()

        chunk(0, 0, drain=False)
        chunk(1, 1)

        def pair(i, carry):
            chunk(2 * i + 2, 0)
            chunk(2 * i + 3, 1)
            return carry

        lax.fori_loop(0, (NCH - 3) // 2, pair, 0)
        chunk(NCH - 1, 0, pre=False)

        wait_s(0)
        plsc.subcore_barrier()
        pltpu.sync_copy(acc_sh.at[pl.ds(rbase, ROWS_PER_TILE)],
                        out_hbm.at[c, pl.ds(rbase, ROWS_PER_TILE)])

    return pl.kernel(
        body_fn,
        mesh=_sc_mesh(),
        compiler_params=_SC_PARAMS,
        out_type=jax.ShapeDtypeStruct((NC, N_PAD, D), jnp.float32),
        scratch_types=(
            [pltpu.VMEM((E_PER_W,), jnp.int32)]
            + [pltpu.VMEM((NCH, CH), jnp.int32)]
            + [pltpu.VMEM((CH, D), jnp.float32)] * 2
            + [pltpu.VMEM_SHARED((N_PAD, D), jnp.float32)]
            + [pltpu.SemaphoreType.DMA] * 4
        ),
    )


_agg128 = _make_agg(D_HID)
_agg48 = _make_agg(D2P)


@functools.partial(
    pl.kernel,
    mesh=_sc_mesh(),
    compiler_params=_SC_PARAMS,
    out_type=jax.ShapeDtypeStruct((NC, N_PAD, 16), jnp.float32),
    scratch_types=(
        [pltpu.VMEM((CH, 16), jnp.float32)]
        + [pltpu.VMEM((CH,), jnp.int32)] * (2 * KD)
        + [pltpu.VMEM_SHARED((N_PAD, 16), jnp.float32)]
        + [pltpu.SemaphoreType.DMA] * 2
    ),
)
def _deg_pass(dst_hbm, ones_hbm, zeros_hbm, out_hbm,
              ones_v, d00, d01, d02, d03, d04, d10, d11, d12, d13, d14,
              acc_sh, ssem0, ssem1):
    c = lax.axis_index("c")
    s = lax.axis_index("s")
    wid = c * NS + s
    base = wid * E_PER_W
    rbase = s * ROWS_PER_TILE

    dstb = ((d00, d01, d02, d03, d04), (d10, d11, d12, d13, d14))
    ssem = (ssem0, ssem1)

    def idx_load(g, b):
        off = base + g * GSZD
        for k in range(KD):
            pltpu.sync_copy(dst_hbm.at[pl.ds(off + k * CH, CH)], dstb[b][k])

    def fire_s(b):
        for k in range(KD):
            pltpu.async_copy(ones_v, acc_sh.at[dstb[b][k]], ssem[b],
                             add=True)

    def wait_s(b):
        for k in range(KD):
            pltpu.make_async_copy(ones_v, acc_sh.at[dstb[b][k]],
                                  ssem[b]).wait()

    pltpu.sync_copy(ones_hbm, ones_v)
    idx_load(0, 0)
    pltpu.sync_copy(zeros_hbm.at[pl.ds(rbase, ROWS_PER_TILE)],
                    acc_sh.at[pl.ds(rbase, ROWS_PER_TILE)])
    plsc.subcore_barrier()

    fire_s(0)
    idx_load(1, 1)

    def group(g, b):
        nb = 1 - b
        fire_s(b)
        wait_s(nb)
        idx_load(g + 1, nb)

    group(1, 1)

    def pair(t, carry):
        group(2 * t + 2, 0)
        group(2 * t + 3, 1)
        return carry

    lax.fori_loop(0, (NGD - 3) // 2, pair, 0)

    fire_s(0)
    wait_s(1)
    wait_s(0)
    plsc.subcore_barrier()
    pltpu.sync_copy(acc_sh.at[pl.ds(rbase, ROWS_PER_TILE)],
                    out_hbm.at[c, pl.ds(rbase, ROWS_PER_TILE)])


_BLK = 1000


def _y1_body(parts_ref, x_ref, w1_ref, dinv_ref, y1_ref):
    deg = parts_ref[0][:, :1] + parts_ref[1][:, :1] + 1.0
    dinv = lax.rsqrt(jnp.clip(deg, 1.0, None))
    dinv_ref[...] = jnp.broadcast_to(dinv, (_BLK, 16))
    y1_ref[...] = jnp.dot(x_ref[...], w1_ref[...],
                          preferred_element_type=jnp.float32) * dinv


def _layer2_body(p_ref, y1_ref, dinv_ref, b1_ref, w2_ref, y2_ref):
    dinv = dinv_ref[:, :1]
    z = p_ref[0] + p_ref[1] + y1_ref[...]
    h = jnp.maximum(z * dinv + b1_ref[...], 0.0)
    y2_ref[...] = jnp.dot(h, w2_ref[...],
                          preferred_element_type=jnp.float32) * dinv


def _epilogue_body(p_ref, y2_ref, dinv_ref, b2_ref, o_ref):
    dinv = dinv_ref[:, :1]
    z = p_ref[0] + p_ref[1] + y2_ref[...]
    o_ref[...] = z * dinv + b2_ref[...]


def kernel(x, edge_index, W1, b1, W2, b2):
    f32 = jnp.float32
    src = edge_index[0]
    dst = edge_index[1]
    ones16 = jnp.ones((CH, 16), f32)
    zeros16 = jnp.zeros((N_PAD, 16), f32)
    zeros48 = jnp.zeros((N_PAD, D2P), f32)
    zeros128 = jnp.zeros((N_PAD, D_HID), f32)
    W2p = jnp.zeros((D_HID, D2P), f32).at[:, :N_CLS].set(W2)
    b1r = b1.reshape(1, D_HID)
    b2p = jnp.zeros((1, D2P), f32).at[0, :N_CLS].set(b2)

    grid = N // _BLK

    deg_parts = _deg_pass(dst, ones16, zeros16)

    dinv16, y1 = pl.pallas_call(
        _y1_body,
        grid=(grid,),
        in_specs=[pl.BlockSpec((NC, _BLK, 16), lambda i: (0, i, 0)),
                  pl.BlockSpec((_BLK, D_IN), lambda i: (i, 0)),
                  pl.BlockSpec((D_IN, D_HID), lambda i: (0, 0))],
        out_specs=[pl.BlockSpec((_BLK, 16), lambda i: (i, 0)),
                   pl.BlockSpec((_BLK, D_HID), lambda i: (i, 0))],
        out_shape=[jax.ShapeDtypeStruct((N, 16), f32),
                   jax.ShapeDtypeStruct((N, D_HID), f32)],
    )(deg_parts, x, W1)

    dst3 = dst.reshape(NW, E_PER_W // CH, CH)

    p1 = _agg128(y1, src, dst3, zeros128)

    y2 = pl.pallas_call(
        _layer2_body,
        grid=(grid,),
        in_specs=[pl.BlockSpec((NC, _BLK, D_HID), lambda i: (0, i, 0)),
                  pl.BlockSpec((_BLK, D_HID), lambda i: (i, 0)),
                  pl.BlockSpec((_BLK, 16), lambda i: (i, 0)),
                  pl.BlockSpec((1, D_HID), lambda i: (0, 0)),
                  pl.BlockSpec((D_HID, D2P), lambda i: (0, 0))],
        out_specs=pl.BlockSpec((_BLK, D2P), lambda i: (i, 0)),
        out_shape=jax.ShapeDtypeStruct((N, D2P), f32),
    )(p1, y1, dinv16, b1r, W2p)

    p2 = _agg48(y2, src, dst3, zeros48)

    outp = pl.pallas_call(
        _epilogue_body,
        grid=(grid,),
        in_specs=[pl.BlockSpec((NC, _BLK, D2P), lambda i: (0, i, 0)),
                  pl.BlockSpec((_BLK, D2P), lambda i: (i, 0)),
                  pl.BlockSpec((_BLK, 16), lambda i: (i, 0)),
                  pl.BlockSpec((1, D2P), lambda i: (0, 0))],
        out_specs=pl.BlockSpec((_BLK, D2P), lambda i: (i, 0)),
        out_shape=jax.ShapeDtypeStruct((N, D2P), f32),
    )(p2, y2, dinv16, b2p)

    return outp[:, :N_CLS]


# speedup vs baseline: 1.9989x; 1.1221x over previous
import functools
import jax
import jax.numpy as jnp
from jax import lax
from jax.experimental import pallas as pl
from jax.experimental.pallas import tpu as pltpu
from jax.experimental.pallas import tpu_sc as plsc

N = 10000
E = 320000
D_IN = 128
D_HID = 128
N_CLS = 40
D2P = 48

NC = 2
NS = 16
NW = NC * NS
E_PER_W = E // NW
CH = 80
KD = 5
GSZD = KD * CH
NGD = E_PER_W // GSZD
N_PAD = 10240
ROWS_PER_TILE = N_PAD // NS


def _sc_mesh():
    return plsc.VectorSubcoreMesh(core_axis_name="c", subcore_axis_name="s")


_SC_PARAMS = pltpu.CompilerParams(use_tc_tiling_on_sc=False)


def _make_agg(D):
    NCH = E_PER_W // CH

    def body_fn(y_hbm, src_hbm, dst_hbm, zeros_hbm, out_hbm,
                srcb, dstb, rows0, rows1,
                acc_sh, gsem0, gsem1, ssem0, ssem1):
        c = lax.axis_index("c")
        s = lax.axis_index("s")
        wid = c * NS + s
        base = wid * E_PER_W
        rbase = s * ROWS_PER_TILE

        rows = (rows0, rows1)
        gsem = (gsem0, gsem1)
        ssem = (ssem0, ssem1)

        def fire_g(t, b):
            pltpu.async_copy(y_hbm.at[srcb.at[pl.ds(t * CH, CH)]],
                             rows[b], gsem[b])

        def wait_g(b):
            pltpu.make_async_copy(y_hbm.at[srcb.at[pl.ds(0, CH)]],
                                  rows[b], gsem[b]).wait()

        def wait_s(b):
            pltpu.make_async_copy(rows[b], acc_sh.at[dstb.at[0]],
                                  ssem[b]).wait()

        def chunk(t, b, drain=True, pre=True):
            wait_g(b)
            pltpu.async_copy(rows[b], acc_sh.at[dstb.at[t]], ssem[b],
                             add=True)
            if drain:
                wait_s(1 - b)
            if pre:
                fire_g(t + 1, 1 - b)

        pltpu.sync_copy(src_hbm.at[pl.ds(base, E_PER_W)], srcb)
        pltpu.sync_copy(dst_hbm.at[wid], dstb)
        fire_g(0, 0)
        pltpu.sync_copy(zeros_hbm.at[pl.ds(rbase, ROWS_PER_TILE)],
                        acc_sh.at[pl.ds(rbase, ROWS_PER_TILE)])
        plsc.subcore_barrier()

        chunk(0, 0, drain=False)
        chunk(1, 1)

        def pair(i, carry):
            chunk(2 * i + 2, 0)
            chunk(2 * i + 3, 1)
            return carry

        lax.fori_loop(0, (NCH - 3) // 2, pair, 0)
        chunk(NCH - 1, 0, pre=False)

        wait_s(0)
        plsc.subcore_barrier()
        pltpu.sync_copy(acc_sh.at[pl.ds(rbase, ROWS_PER_TILE)],
                        out_hbm.at[c, pl.ds(rbase, ROWS_PER_TILE)])

    return pl.kernel(
        body_fn,
        mesh=_sc_mesh(),
        compiler_params=_SC_PARAMS,
        out_type=jax.ShapeDtypeStruct((NC, N_PAD, D), jnp.float32),
        scratch_types=(
            [pltpu.VMEM((E_PER_W,), jnp.int32)]
            + [pltpu.VMEM((NCH, CH), jnp.int32)]
            + [pltpu.VMEM((CH, D), jnp.float32)] * 2
            + [pltpu.VMEM_SHARED((N_PAD, D), jnp.float32)]
            + [pltpu.SemaphoreType.DMA] * 4
        ),
    )


_agg128 = _make_agg(D_HID)
_agg48 = _make_agg(D2P)


@functools.partial(
    pl.kernel,
    mesh=_sc_mesh(),
    compiler_params=_SC_PARAMS,
    out_type=jax.ShapeDtypeStruct((NC, N_PAD, 16), jnp.float32),
    scratch_types=(
        [pltpu.VMEM((CH, 16), jnp.float32)]
        + [pltpu.VMEM((E_PER_W // CH, CH), jnp.int32)]
        + [pltpu.VMEM_SHARED((N_PAD, 16), jnp.float32)]
        + [pltpu.SemaphoreType.DMA] * 2
    ),
)
def _deg_pass(dst_hbm, ones_hbm, zeros_hbm, out_hbm,
              ones_v, dstb, acc_sh, ssem0, ssem1):
    c = lax.axis_index("c")
    s = lax.axis_index("s")
    wid = c * NS + s
    rbase = s * ROWS_PER_TILE

    ssem = (ssem0, ssem1)

    def fire_s(g, b):
        for k in range(KD):
            pltpu.async_copy(ones_v, acc_sh.at[dstb.at[g * KD + k]],
                             ssem[b], add=True)

    def wait_s(b):
        for k in range(KD):
            pltpu.make_async_copy(ones_v, acc_sh.at[dstb.at[0]],
                                  ssem[b]).wait()

    pltpu.sync_copy(ones_hbm, ones_v)
    pltpu.sync_copy(dst_hbm.at[wid], dstb)
    pltpu.sync_copy(zeros_hbm.at[pl.ds(rbase, ROWS_PER_TILE)],
                    acc_sh.at[pl.ds(rbase, ROWS_PER_TILE)])
    plsc.subcore_barrier()

    fire_s(0, 0)
    fire_s(1, 1)
    wait_s(0)

    def pair(t, carry):
        fire_s(2 * t + 2, 0)
        wait_s(1)
        fire_s(2 * t + 3, 1)
        wait_s(0)
        return carry

    lax.fori_loop(0, (NGD - 3) // 2, pair, 0)

    fire_s(NGD - 1, 0)
    wait_s(1)
    wait_s(0)
    plsc.subcore_barrier()
    pltpu.sync_copy(acc_sh.at[pl.ds(rbase, ROWS_PER_TILE)],
                    out_hbm.at[c, pl.ds(rbase, ROWS_PER_TILE)])


_BLK = 1000


def _y1_body(parts_ref, x_ref, w1_ref, dinv_ref, y1_ref):
    deg = parts_ref[0][:, :1] + parts_ref[1][:, :1] + 1.0
    dinv = lax.rsqrt(jnp.clip(deg, 1.0, None))
    dinv_ref[...] = jnp.broadcast_to(dinv, (_BLK, 16))
    y1_ref[...] = jnp.dot(x_ref[...], w1_ref[...],
                          preferred_element_type=jnp.float32) * dinv


def _layer2_body(p_ref, y1_ref, dinv_ref, b1_ref, w2_ref, y2_ref):
    dinv = dinv_ref[:, :1]
    z = p_ref[0] + p_ref[1] + y1_ref[...]
    h = jnp.maximum(z * dinv + b1_ref[...], 0.0)
    y2_ref[...] = jnp.dot(h, w2_ref[...],
                          preferred_element_type=jnp.float32) * dinv


def _epilogue_body(p_ref, y2_ref, dinv_ref, b2_ref, o_ref):
    dinv = dinv_ref[:, :1]
    z = p_ref[0] + p_ref[1] + y2_ref[...]
    o_ref[...] = z * dinv + b2_ref[...]


def kernel(x, edge_index, W1, b1, W2, b2):
    f32 = jnp.float32
    src = edge_index[0]
    dst = edge_index[1]
    ones16 = jnp.ones((CH, 16), f32)
    zeros16 = jnp.zeros((N_PAD, 16), f32)
    zeros48 = jnp.zeros((N_PAD, D2P), f32)
    zeros128 = jnp.zeros((N_PAD, D_HID), f32)
    W2p = jnp.zeros((D_HID, D2P), f32).at[:, :N_CLS].set(W2)
    b1r = b1.reshape(1, D_HID)
    b2p = jnp.zeros((1, D2P), f32).at[0, :N_CLS].set(b2)

    grid = N // _BLK
    dst3 = dst.reshape(NW, E_PER_W // CH, CH)

    deg_parts = _deg_pass(dst3, ones16, zeros16)

    dinv16, y1 = pl.pallas_call(
        _y1_body,
        grid=(grid,),
        in_specs=[pl.BlockSpec((NC, _BLK, 16), lambda i: (0, i, 0)),
                  pl.BlockSpec((_BLK, D_IN), lambda i: (i, 0)),
                  pl.BlockSpec((D_IN, D_HID), lambda i: (0, 0))],
        out_specs=[pl.BlockSpec((_BLK, 16), lambda i: (i, 0)),
                   pl.BlockSpec((_BLK, D_HID), lambda i: (i, 0))],
        out_shape=[jax.ShapeDtypeStruct((N, 16), f32),
                   jax.ShapeDtypeStruct((N, D_HID), f32)],
    )(deg_parts, x, W1)

    p1 = _agg128(y1, src, dst3, zeros128)

    y2 = pl.pallas_call(
        _layer2_body,
        grid=(grid,),
        in_specs=[pl.BlockSpec((NC, _BLK, D_HID), lambda i: (0, i, 0)),
                  pl.BlockSpec((_BLK, D_HID), lambda i: (i, 0)),
                  pl.BlockSpec((_BLK, 16), lambda i: (i, 0)),
                  pl.BlockSpec((1, D_HID), lambda i: (0, 0)),
                  pl.BlockSpec((D_HID, D2P), lambda i: (0, 0))],
        out_specs=pl.BlockSpec((_BLK, D2P), lambda i: (i, 0)),
        out_shape=jax.ShapeDtypeStruct((N, D2P), f32),
    )(p1, y1, dinv16, b1r, W2p)

    p2 = _agg48(y2, src, dst3, zeros48)

    outp = pl.pallas_call(
        _epilogue_body,
        grid=(grid,),
        in_specs=[pl.BlockSpec((NC, _BLK, D2P), lambda i: (0, i, 0)),
                  pl.BlockSpec((_BLK, D2P), lambda i: (i, 0)),
                  pl.BlockSpec((_BLK, 16), lambda i: (i, 0)),
                  pl.BlockSpec((1, D2P), lambda i: (0, 0))],
        out_specs=pl.BlockSpec((_BLK, D2P), lambda i: (i, 0)),
        out_shape=jax.ShapeDtypeStruct((N, D2P), f32),
    )(p2, y2, dinv16, b2p)

    return outp[:, :N_CLS]
